# baseline (speedup 1.0000x reference)
import functools

import jax
import jax.numpy as jnp
from jax.experimental import pallas as pl
from jax.experimental.pallas import tpu as pltpu

_MAX_VAL = 16383
_S = 16384
_D = 4096
_CHUNK = 128


def _pipeline_kernel(maskT_ref, out_ref, hi_ref, lo_ref):
    i = pl.program_id(0)

    @pl.when(i == 0)
    def _compute_idx():
        m = maskT_ref[...].astype(jnp.float32)
        row = jax.lax.broadcasted_iota(jnp.int32, (_CHUNK, _CHUNK), 0)
        col = jax.lax.broadcasted_iota(jnp.int32, (_CHUNK, _CHUNK), 1)
        l_incl = (col <= row).astype(jnp.float32)
        u_strict = (row < col).astype(jnp.float32)
        csT = jnp.dot(l_incl, m, preferred_element_type=jnp.float32)
        prefT = jnp.dot(csT[_CHUNK - 1:_CHUNK, :], u_strict,
                        preferred_element_type=jnp.float32)
        idxT = csT + prefT - 1.0
        idxT = jnp.clip(idxT, 0.0, float(_MAX_VAL))
        hi = jnp.floor(idxT * (1.0 / _CHUNK))
        hi_ref[...] = hi
        lo_ref[...] = idxT - hi * float(_CHUNK)

    sub = jax.lax.broadcasted_iota(jnp.int32, (_CHUNK, 1), 0)
    onehot = (sub == i).astype(jnp.float32)
    hi_col = jnp.dot(hi_ref[...], onehot, preferred_element_type=jnp.float32)
    lo_col = jnp.dot(lo_ref[...], onehot, preferred_element_type=jnp.float32)
    colv = hi_col * float(_CHUNK) + lo_col
    out_ref[...] = jnp.broadcast_to(colv.astype(jnp.int32), (_CHUNK, _D))


@functools.partial(jax.jit, static_argnames=())
def kernel(mask_1d, inputs_embeds_row):
    del inputs_embeds_row
    maskT = mask_1d.astype(jnp.int32).reshape(_S // _CHUNK, _CHUNK).T
    grid = _S // _CHUNK
    return pl.pallas_call(
        _pipeline_kernel,
        grid=(grid,),
        in_specs=[pl.BlockSpec((_CHUNK, _CHUNK), lambda i: (0, 0))],
        out_specs=pl.BlockSpec((_CHUNK, _D), lambda i: (i, 0)),
        out_shape=jax.ShapeDtypeStruct((_S, _D), jnp.int32),
        scratch_shapes=[pltpu.VMEM((_CHUNK, _CHUNK), jnp.float32),
                        pltpu.VMEM((_CHUNK, _CHUNK), jnp.float32)],
    )(maskT)


# speedup vs baseline: 1.3058x; 1.3058x over previous
import functools

import jax
import jax.numpy as jnp
from jax.experimental import pallas as pl
from jax.experimental.pallas import tpu as pltpu

_MAX_VAL = 16383
_S = 16384
_D = 4096
_CHUNK = 128
_ROWS_PER_BLOCK = 512
_COLS_PER_BLOCK = _ROWS_PER_BLOCK // _CHUNK


def _pipeline_kernel(maskT_ref, out_ref, hi_ref, lo_ref):
    i = pl.program_id(0)

    @pl.when(i == 0)
    def _compute_idx():
        m = maskT_ref[...].astype(jnp.float32)
        row = jax.lax.broadcasted_iota(jnp.int32, (_CHUNK, _CHUNK), 0)
        col = jax.lax.broadcasted_iota(jnp.int32, (_CHUNK, _CHUNK), 1)
        l_incl = (col <= row).astype(jnp.float32)
        u_strict = (row < col).astype(jnp.float32)
        csT = jnp.dot(l_incl, m, preferred_element_type=jnp.float32)
        prefT = jnp.dot(csT[_CHUNK - 1:_CHUNK, :], u_strict,
                        preferred_element_type=jnp.float32)
        idxT = csT + prefT - 1.0
        idxT = jnp.clip(idxT, 0.0, float(_MAX_VAL))
        hi = jnp.floor(idxT * (1.0 / _CHUNK))
        hi_ref[...] = hi
        lo_ref[...] = idxT - hi * float(_CHUNK)

    sub = jax.lax.broadcasted_iota(jnp.int32, (_CHUNK, 1), 0)
    for j in range(_COLS_PER_BLOCK):
        onehot = (sub == i * _COLS_PER_BLOCK + j).astype(jnp.float32)
        hi_col = jnp.dot(hi_ref[...], onehot,
                         preferred_element_type=jnp.float32)
        lo_col = jnp.dot(lo_ref[...], onehot,
                         preferred_element_type=jnp.float32)
        colv = hi_col * float(_CHUNK) + lo_col
        out_ref[pl.ds(j * _CHUNK, _CHUNK), :] = jnp.broadcast_to(
            colv.astype(jnp.int32), (_CHUNK, _D))


@functools.partial(jax.jit, static_argnames=())
def kernel(mask_1d, inputs_embeds_row):
    del inputs_embeds_row
    maskT = mask_1d.astype(jnp.int32).reshape(_S // _CHUNK, _CHUNK).T
    grid = _S // _ROWS_PER_BLOCK
    return pl.pallas_call(
        _pipeline_kernel,
        grid=(grid,),
        in_specs=[pl.BlockSpec((_CHUNK, _CHUNK), lambda i: (0, 0))],
        out_specs=pl.BlockSpec((_ROWS_PER_BLOCK, _D), lambda i: (i, 0)),
        out_shape=jax.ShapeDtypeStruct((_S, _D), jnp.int32),
        scratch_shapes=[pltpu.VMEM((_CHUNK, _CHUNK), jnp.float32),
                        pltpu.VMEM((_CHUNK, _CHUNK), jnp.float32)],
    )(maskT)
